# baseline (speedup 1.0000x reference)
from functools import partial

import jax
import jax.numpy as jnp
from jax.experimental import pallas as pl

_NUM_EXPERTS = 8
_TOP_K = 2
_NSPLIT = 1
_T = 4096


def _route_scores(scores):
    iota = jax.lax.broadcasted_iota(jnp.int32, scores.shape, 0)
    m1 = jnp.max(scores, axis=0, keepdims=True)
    i1 = jnp.min(jnp.where(scores == m1, iota, _NUM_EXPERTS),
                 axis=0, keepdims=True)
    masked = jnp.where(iota == i1, -jnp.inf, scores)
    m2 = jnp.max(masked, axis=0, keepdims=True)
    i2 = jnp.min(jnp.where(masked == m2, iota, _NUM_EXPERTS),
                 axis=0, keepdims=True)
    idx = jnp.concatenate([i1, i2], axis=0)

    e = jnp.exp(m2 - m1)
    w1 = 1.0 / (1.0 + e)
    wgt = jnp.concatenate([w1, 1.0 - w1], axis=0)

    p = jnp.exp(scores - m1)
    p = p / jnp.sum(p, axis=0, keepdims=True)
    usage = jnp.sum(p, axis=1, keepdims=True)
    return idx, wgt, usage


def _router_block(*refs, nsteps, inv_ntokens):
    x_refs = refs[:_NSPLIT]
    w_ref = refs[_NSPLIT]
    idx_refs = refs[_NSPLIT + 1:2 * _NSPLIT + 1]
    wgt_refs = refs[2 * _NSPLIT + 1:3 * _NSPLIT + 1]
    usage_ref = refs[3 * _NSPLIT + 1]
    aux_ref = refs[3 * _NSPLIT + 2]

    i = pl.program_id(0)
    w = w_ref[...]
    part = None
    for x_ref, idx_ref, wgt_ref in zip(x_refs, idx_refs, wgt_refs):
        scores = jax.lax.dot_general(
            w, x_ref[...], (((1,), (1,)), ((), ())),
            preferred_element_type=jnp.float32)
        idx, wgt, usage = _route_scores(scores)
        idx_ref[...] = idx.T
        wgt_ref[...] = wgt.T
        part = usage if part is None else part + usage

    @pl.when(i == 0)
    def _():
        usage_ref[...] = part

    @pl.when(i != 0)
    def _():
        usage_ref[...] += part

    @pl.when(i == nsteps - 1)
    def _():
        u = usage_ref[...] * inv_ntokens
        aux_ref[...] = _NUM_EXPERTS * jnp.sum(u * u, axis=0, keepdims=True)


@jax.jit
def kernel(x, W):
    B, S, D = x.shape
    N = B * S
    M = N // _NSPLIT
    xf = x.reshape(N, D)
    xs = [xf] * _NSPLIT
    nsteps = M // _T

    x_specs = [pl.BlockSpec((_T, D), partial(lambda j, i: (j * nsteps + i, 0), j))
               for j in range(_NSPLIT)]
    tk_specs = [pl.BlockSpec((_T, _TOP_K), lambda i: (i, 0))
                for _ in range(_NSPLIT)]
    tk_shapes = [jax.ShapeDtypeStruct((M, _TOP_K), dt)
                 for dt in (jnp.int32, jnp.float32) for _ in range(_NSPLIT)]

    outs = pl.pallas_call(
        partial(_router_block, nsteps=nsteps, inv_ntokens=1.0 / N),
        grid=(nsteps,),
        in_specs=x_specs + [pl.BlockSpec((_NUM_EXPERTS, D), lambda i: (0, 0))],
        out_specs=tk_specs + tk_specs + [
            pl.BlockSpec((_NUM_EXPERTS, 1), lambda i: (0, 0)),
            pl.BlockSpec((1, 1), lambda i: (0, 0)),
        ],
        out_shape=tk_shapes + [
            jax.ShapeDtypeStruct((_NUM_EXPERTS, 1), jnp.float32),
            jax.ShapeDtypeStruct((1, 1), jnp.float32),
        ],
    )(*xs, W)

    idxs = outs[:_NSPLIT]
    wgts = outs[_NSPLIT:2 * _NSPLIT]
    aux = outs[2 * _NSPLIT + 1]

    idx = jnp.concatenate(list(idxs), axis=0)
    wgt = jnp.concatenate(list(wgts), axis=0)
    return (idx.reshape(B, S, _TOP_K),
            wgt.reshape(B, S, _TOP_K),
            aux[0, 0])


# speedup vs baseline: 1.9002x; 1.9002x over previous
from functools import partial

import jax
import jax.numpy as jnp
from jax.experimental import pallas as pl

_NUM_EXPERTS = 8
_TOP_K = 2
_NSPLIT = 1
_T = 4096


def _route_scores(scores):
    iota = jax.lax.broadcasted_iota(jnp.int32, scores.shape, 0)
    m1 = jnp.max(scores, axis=0, keepdims=True)
    i1 = jnp.min(jnp.where(scores == m1, iota, _NUM_EXPERTS),
                 axis=0, keepdims=True)
    masked = jnp.where(iota == i1, -jnp.inf, scores)
    m2 = jnp.max(masked, axis=0, keepdims=True)
    i2 = jnp.min(jnp.where(masked == m2, iota, _NUM_EXPERTS),
                 axis=0, keepdims=True)
    idx = jnp.concatenate([i1, i2], axis=0)

    e = jnp.exp(m2 - m1)
    w1 = 1.0 / (1.0 + e)
    wgt = jnp.concatenate([w1, 1.0 - w1], axis=0)

    p = jnp.exp(scores - m1)
    p = p / jnp.sum(p, axis=0, keepdims=True)
    usage = jnp.sum(p, axis=1, keepdims=True)
    return idx, wgt, usage


def _router_block(*refs, nsteps, inv_ntokens):
    x_refs = refs[:_NSPLIT]
    w_ref = refs[_NSPLIT]
    idx_refs = refs[_NSPLIT + 1:2 * _NSPLIT + 1]
    wgt_refs = refs[2 * _NSPLIT + 1:3 * _NSPLIT + 1]
    usage_ref = refs[3 * _NSPLIT + 1]
    aux_ref = refs[3 * _NSPLIT + 2]

    i = pl.program_id(0)
    w = w_ref[...]
    part = None
    for x_ref, idx_ref, wgt_ref in zip(x_refs, idx_refs, wgt_refs):
        scores = jax.lax.dot_general(
            w, x_ref[...], (((1,), (1,)), ((), ())),
            preferred_element_type=jnp.float32)
        idx, wgt, usage = _route_scores(scores)
        idx_ref[...] = idx
        wgt_ref[...] = wgt
        part = usage if part is None else part + usage

    @pl.when(i == 0)
    def _():
        usage_ref[...] = part

    @pl.when(i != 0)
    def _():
        usage_ref[...] += part

    @pl.when(i == nsteps - 1)
    def _():
        u = usage_ref[...] * inv_ntokens
        aux_ref[...] = _NUM_EXPERTS * jnp.sum(u * u, axis=0, keepdims=True)


@jax.jit
def kernel(x, W):
    B, S, D = x.shape
    N = B * S
    M = N // _NSPLIT
    xf = x.reshape(N, D)
    xs = [xf] * _NSPLIT
    nsteps = M // _T

    x_specs = [pl.BlockSpec((_T, D), partial(lambda j, i: (j * nsteps + i, 0), j))
               for j in range(_NSPLIT)]
    tk_specs = [pl.BlockSpec((_TOP_K, _T), lambda i: (0, i))
                for _ in range(_NSPLIT)]
    tk_shapes = [jax.ShapeDtypeStruct((_TOP_K, M), dt)
                 for dt in (jnp.int32, jnp.float32) for _ in range(_NSPLIT)]

    outs = pl.pallas_call(
        partial(_router_block, nsteps=nsteps, inv_ntokens=1.0 / N),
        grid=(nsteps,),
        in_specs=x_specs + [pl.BlockSpec((_NUM_EXPERTS, D), lambda i: (0, 0))],
        out_specs=tk_specs + tk_specs + [
            pl.BlockSpec((_NUM_EXPERTS, 1), lambda i: (0, 0)),
            pl.BlockSpec((1, 1), lambda i: (0, 0)),
        ],
        out_shape=tk_shapes + [
            jax.ShapeDtypeStruct((_NUM_EXPERTS, 1), jnp.float32),
            jax.ShapeDtypeStruct((1, 1), jnp.float32),
        ],
    )(*xs, W)

    idxs = outs[:_NSPLIT]
    wgts = outs[_NSPLIT:2 * _NSPLIT]
    aux = outs[2 * _NSPLIT + 1]

    idx = jnp.concatenate([a.T for a in idxs], axis=0)
    wgt = jnp.concatenate([a.T for a in wgts], axis=0)
    return (idx.reshape(B, S, _TOP_K),
            wgt.reshape(B, S, _TOP_K),
            aux[0, 0])
